# baseline (speedup 1.0000x reference)
import functools

import jax
import jax.numpy as jnp
from jax.experimental import pallas as pl
from jax.experimental.pallas import tpu as pltpu

_VMEM_LIMIT = 56 * 1024 * 1024
_BS = 2
_DEPTH = 2


def _fused_kernel(x_ref, w_ref, g_ref, b_ref, o_ref,
                  xb_ref, obuf_ref, ssum_ref, ssq_ref, scale_ref, shift_ref,
                  sem_in_ref, sem_out_ref, *, nsteps, msteps, count, eps):
    j = pl.program_id(0)
    wb = w_ref[...].astype(jnp.bfloat16)

    def _in_copy(k):
        return pltpu.make_async_copy(
            x_ref.at[pl.ds(k * _BS, _BS)],
            xb_ref.at[pl.ds(k * _BS, _BS)],
            sem_in_ref.at[k])

    def _out_copy(src_slot, k):
        return pltpu.make_async_copy(
            obuf_ref.at[src_slot],
            o_ref.at[pl.ds(k * _BS, _BS)],
            sem_out_ref.at[src_slot])

    @pl.when(j == 0)
    def _init():
        ssum_ref[...] = jnp.zeros_like(ssum_ref)
        ssq_ref[...] = jnp.zeros_like(ssq_ref)
        for k in range(nsteps):
            _in_copy(k).start()

    @pl.when(j < nsteps)
    def _ingest():
        _in_copy(j).wait()
        for s in range(_BS):
            xb = xb_ref[j * _BS + s]
            u = jnp.dot(wb, xb, preferred_element_type=jnp.float32)
            ssum_ref[...] += jnp.sum(u, axis=1, keepdims=True)
            ssq_ref[...] += jnp.sum(u * u, axis=1, keepdims=True)

    @pl.when(j == nsteps)
    def _fold():
        mean = ssum_ref[...] * (1.0 / count)
        var = jnp.maximum(ssq_ref[...] * (1.0 / count) - mean * mean, 0.0)
        scale = g_ref[...] * jax.lax.rsqrt(var + jnp.float32(eps))
        scale_ref[...] = scale
        shift_ref[...] = b_ref[...] - mean * scale

    @pl.when(j >= nsteps)
    def _emit():
        k = j - nsteps
        slot = jax.lax.rem(k, _DEPTH)

        @pl.when(k >= _DEPTH)
        def _drain_prev():
            _out_copy(slot, k - _DEPTH).wait()

        for s in range(_BS):
            xb = xb_ref[k * _BS + s]
            u = jnp.dot(wb, xb, preferred_element_type=jnp.float32)
            z = u * scale_ref[...] + shift_ref[...]
            obuf_ref[slot, s] = jnp.where(z >= 0, z, 0.01 * z
                                          ).astype(obuf_ref.dtype)

        _out_copy(slot, k).start()

        @pl.when(k == msteps - 1)
        def _drain_all():
            for d in range(_DEPTH - 1, -1, -1):
                @pl.when(k - d >= 0)
                def _(d=d):
                    _out_copy(jax.lax.rem(k - d, _DEPTH), k - d).wait()


def kernel(x_nchw, w_conv, b_conv, gamma, beta, eps=1e-5):
    N, Cin, H, W = x_nchw.shape
    Cout = w_conv.shape[0]
    P = H * W
    Pp = ((P + 127) // 128) * 128
    del b_conv

    x3 = x_nchw.reshape(N, Cin, P)
    xpad = jnp.pad(x3, ((0, 0), (0, 0), (0, Pp - P))).astype(jnp.bfloat16)
    w2 = w_conv.reshape(Cout, Cin)
    g2 = gamma.reshape(Cout, 1)
    b2 = beta.reshape(Cout, 1)
    count = float(N * P)

    nsteps = N // _BS
    msteps = N // _BS

    opad = pl.pallas_call(
        functools.partial(_fused_kernel, nsteps=nsteps, msteps=msteps,
                          count=count, eps=eps),
        out_shape=jax.ShapeDtypeStruct((N, Cout, Pp), jnp.bfloat16),
        grid=(nsteps + msteps,),
        in_specs=[
            pl.BlockSpec(memory_space=pl.ANY),
            pl.BlockSpec((Cout, Cin), lambda j: (0, 0)),
            pl.BlockSpec((Cout, 1), lambda j: (0, 0)),
            pl.BlockSpec((Cout, 1), lambda j: (0, 0)),
        ],
        out_specs=pl.BlockSpec(memory_space=pl.ANY),
        scratch_shapes=[
            pltpu.VMEM((N, Cin, Pp), jnp.bfloat16),
            pltpu.VMEM((_DEPTH, _BS, Cout, Pp), jnp.bfloat16),
            pltpu.VMEM((Cout, 1), jnp.float32),
            pltpu.VMEM((Cout, 1), jnp.float32),
            pltpu.VMEM((Cout, 1), jnp.float32),
            pltpu.VMEM((Cout, 1), jnp.float32),
            pltpu.SemaphoreType.DMA((nsteps,)),
            pltpu.SemaphoreType.DMA((_DEPTH,)),
        ],
        compiler_params=pltpu.CompilerParams(
            dimension_semantics=("arbitrary",),
            vmem_limit_bytes=_VMEM_LIMIT,
        ),
    )(xpad, w2, g2, b2)

    return opad[:, :, :P].astype(jnp.float32).reshape(N, Cout, H, W)


# speedup vs baseline: 1.2941x; 1.2941x over previous
import functools

import jax
import jax.numpy as jnp
from jax.experimental import pallas as pl
from jax.experimental.pallas import tpu as pltpu

_VMEM_LIMIT = 56 * 1024 * 1024


def _fused_kernel(x_ref, w_ref, g_ref, b_ref, o_ref,
                  xb_ref, obuf_ref, ssum_ref, ssq_ref, scale_ref, shift_ref,
                  sem_ref, *, bs_in, bs_out, nsteps, msteps, count, eps):
    j = pl.program_id(0)
    wb = w_ref[...].astype(jnp.bfloat16)

    @pl.when(j == 0)
    def _init():
        ssum_ref[...] = jnp.zeros_like(ssum_ref)
        ssq_ref[...] = jnp.zeros_like(ssq_ref)

    @pl.when(j < nsteps)
    def _ingest():
        for s in range(bs_in):
            xb = x_ref[s].astype(jnp.bfloat16)
            xb_ref[pl.ds(j * bs_in + s, 1)] = xb[None]
            u = jnp.dot(wb, xb, preferred_element_type=jnp.float32)
            ssum_ref[...] += jnp.sum(u, axis=1, keepdims=True)
            ssq_ref[...] += jnp.sum(u * u, axis=1, keepdims=True)

    @pl.when(j == nsteps)
    def _fold():
        mean = ssum_ref[...] * (1.0 / count)
        var = jnp.maximum(ssq_ref[...] * (1.0 / count) - mean * mean, 0.0)
        scale = g_ref[...] * jax.lax.rsqrt(var + jnp.float32(eps))
        scale_ref[...] = scale
        shift_ref[...] = b_ref[...] - mean * scale

    @pl.when(j >= nsteps)
    def _emit():
        jj = j - nsteps
        depth = obuf_ref.shape[0]
        slot = jax.lax.rem(jj, depth)

        def _copy(src_slot, dst_step):
            return pltpu.make_async_copy(
                obuf_ref.at[src_slot],
                o_ref.at[pl.ds(dst_step * bs_out, bs_out)],
                sem_ref.at[src_slot])

        @pl.when(jj >= depth)
        def _drain_prev():
            _copy(slot, jj - depth).wait()

        for s in range(bs_out):
            xb = xb_ref[jj * bs_out + s]
            u = jnp.dot(wb, xb, preferred_element_type=jnp.float32)
            z = u * scale_ref[...] + shift_ref[...]
            obuf_ref[slot, s] = jnp.where(z >= 0, z, 0.01 * z)

        _copy(slot, jj).start()

        @pl.when(jj == msteps - 1)
        def _drain_all():
            for d in range(depth - 1, -1, -1):
                @pl.when(jj - d >= 0)
                def _(d=d):
                    _copy(jax.lax.rem(jj - d, depth), jj - d).wait()


def kernel(x_nchw, w_conv, b_conv, gamma, beta, eps=1e-5):
    N, Cin, H, W = x_nchw.shape
    Cout = w_conv.shape[0]
    P = H * W
    del b_conv

    x3 = x_nchw.reshape(N, Cin, P)
    w2 = w_conv.reshape(Cout, Cin)
    g2 = gamma.reshape(Cout, 1)
    b2 = beta.reshape(Cout, 1)
    count = float(N * P)

    bs_in = 2
    bs_out = 1
    depth = 4
    nsteps = N // bs_in
    msteps = N // bs_out

    out3 = pl.pallas_call(
        functools.partial(_fused_kernel, bs_in=bs_in, bs_out=bs_out,
                          nsteps=nsteps, msteps=msteps, count=count, eps=eps),
        out_shape=jax.ShapeDtypeStruct((N, Cout, P), x_nchw.dtype),
        grid=(nsteps + msteps,),
        in_specs=[
            pl.BlockSpec((bs_in, Cin, P),
                         lambda j: (jnp.minimum(j, nsteps - 1), 0, 0)),
            pl.BlockSpec((Cout, Cin), lambda j: (0, 0)),
            pl.BlockSpec((Cout, 1), lambda j: (0, 0)),
            pl.BlockSpec((Cout, 1), lambda j: (0, 0)),
        ],
        out_specs=pl.BlockSpec(memory_space=pl.ANY),
        scratch_shapes=[
            pltpu.VMEM((N, Cin, P), jnp.bfloat16),
            pltpu.VMEM((depth, bs_out, Cout, P), jnp.float32),
            pltpu.VMEM((Cout, 1), jnp.float32),
            pltpu.VMEM((Cout, 1), jnp.float32),
            pltpu.VMEM((Cout, 1), jnp.float32),
            pltpu.VMEM((Cout, 1), jnp.float32),
            pltpu.SemaphoreType.DMA((depth,)),
        ],
        compiler_params=pltpu.CompilerParams(
            dimension_semantics=("arbitrary",),
            vmem_limit_bytes=_VMEM_LIMIT,
        ),
    )(x3, w2, g2, b2)

    return out3.reshape(N, Cout, H, W)


# speedup vs baseline: 1.3190x; 1.0193x over previous
import functools

import jax
import jax.numpy as jnp
from jax.experimental import pallas as pl
from jax.experimental.pallas import tpu as pltpu

_VMEM_LIMIT = 56 * 1024 * 1024


def _fused_kernel(x_ref, w_ref, g_ref, b_ref, o_ref,
                  xb_ref, obuf_ref, ssum_ref, ssq_ref, scale_ref, shift_ref,
                  sem_ref, *, bs_in, bs_out, nsteps, msteps, count, eps):
    j = pl.program_id(0)
    wb = w_ref[...].astype(jnp.bfloat16)

    @pl.when(j == 0)
    def _init():
        ssum_ref[...] = jnp.zeros_like(ssum_ref)
        ssq_ref[...] = jnp.zeros_like(ssq_ref)

    @pl.when(j < nsteps)
    def _ingest():
        for s in range(bs_in):
            xb = x_ref[s].astype(jnp.bfloat16)
            xb_ref[pl.ds(j * bs_in + s, 1)] = xb[None]
            u = jnp.dot(wb, xb, preferred_element_type=jnp.float32)
            ssum_ref[...] += jnp.sum(u, axis=1, keepdims=True)
            ssq_ref[...] += jnp.sum(u * u, axis=1, keepdims=True)

    @pl.when(j == nsteps)
    def _fold():
        mean = ssum_ref[...] * (1.0 / count)
        var = jnp.maximum(ssq_ref[...] * (1.0 / count) - mean * mean, 0.0)
        scale = g_ref[...] * jax.lax.rsqrt(var + jnp.float32(eps))
        scale_ref[...] = scale
        shift_ref[...] = b_ref[...] - mean * scale

    @pl.when(j >= nsteps)
    def _emit():
        jj = j - nsteps
        depth = obuf_ref.shape[0]
        slot = jax.lax.rem(jj, depth)

        def _copy(src_slot, dst_step):
            return pltpu.make_async_copy(
                obuf_ref.at[src_slot],
                o_ref.at[pl.ds(dst_step * bs_out, bs_out)],
                sem_ref.at[src_slot])

        @pl.when(jj >= depth)
        def _drain_prev():
            _copy(slot, jj - depth).wait()

        for s in range(bs_out):
            xb = xb_ref[jj * bs_out + s]
            u = jnp.dot(wb, xb, preferred_element_type=jnp.float32)
            z = u * scale_ref[...] + shift_ref[...]
            obuf_ref[slot, s] = jnp.where(z >= 0, z, 0.01 * z)

        _copy(slot, jj).start()

        @pl.when(jj == msteps - 1)
        def _drain_all():
            for d in range(depth - 1, -1, -1):
                @pl.when(jj - d >= 0)
                def _(d=d):
                    _copy(jax.lax.rem(jj - d, depth), jj - d).wait()


def kernel(x_nchw, w_conv, b_conv, gamma, beta, eps=1e-5):
    N, Cin, H, W = x_nchw.shape
    Cout = w_conv.shape[0]
    P = H * W
    del b_conv

    x3 = x_nchw.reshape(N, Cin, P)
    w2 = w_conv.reshape(Cout, Cin)
    g2 = gamma.reshape(Cout, 1)
    b2 = beta.reshape(Cout, 1)
    count = float(N * P)

    bs_in = 4
    bs_out = 1
    depth = 4
    nsteps = N // bs_in
    msteps = N // bs_out

    out3 = pl.pallas_call(
        functools.partial(_fused_kernel, bs_in=bs_in, bs_out=bs_out,
                          nsteps=nsteps, msteps=msteps, count=count, eps=eps),
        out_shape=jax.ShapeDtypeStruct((N, Cout, P), x_nchw.dtype),
        grid=(nsteps + msteps,),
        in_specs=[
            pl.BlockSpec((bs_in, Cin, P),
                         lambda j: (jnp.minimum(j, nsteps - 1), 0, 0)),
            pl.BlockSpec((Cout, Cin), lambda j: (0, 0)),
            pl.BlockSpec((Cout, 1), lambda j: (0, 0)),
            pl.BlockSpec((Cout, 1), lambda j: (0, 0)),
        ],
        out_specs=pl.BlockSpec(memory_space=pl.ANY),
        scratch_shapes=[
            pltpu.VMEM((N, Cin, P), jnp.bfloat16),
            pltpu.VMEM((depth, bs_out, Cout, P), jnp.float32),
            pltpu.VMEM((Cout, 1), jnp.float32),
            pltpu.VMEM((Cout, 1), jnp.float32),
            pltpu.VMEM((Cout, 1), jnp.float32),
            pltpu.VMEM((Cout, 1), jnp.float32),
            pltpu.SemaphoreType.DMA((depth,)),
        ],
        compiler_params=pltpu.CompilerParams(
            dimension_semantics=("arbitrary",),
            vmem_limit_bytes=_VMEM_LIMIT,
        ),
    )(x3, w2, g2, b2)

    return out3.reshape(N, Cout, H, W)
